# Initial kernel scaffold; baseline (speedup 1.0000x reference)
import jax
import jax.numpy as jnp
from jax.experimental import pallas as pl


def kernel(updates, mask):
    raise NotImplementedError("write your pallas kernel here")


# speedup vs baseline: 20.8660x; 20.8660x over previous
import functools

import jax
import jax.numpy as jnp
from jax import lax
from jax.experimental import pallas as pl
from jax.experimental.pallas import tpu as pltpu
from jax.experimental.pallas import tpu_sc as plsc

_B, _H, _W, _C = 8, 112, 112, 96
_UP = (2, 2)
_OH, _OW = _H * _UP[0], _W * _UP[1]
_R = _OH * _OW
_NPIX = _H * _W
_NPLANES = _B * _C
_NC, _NS = 2, 16
_NW = _NC * _NS
_PPW = _NPLANES // _NW
_LANES = 16


def _unpool_body(upd_hbm, msk_hbm, out_hbm, plane, updv, mskv, sem_u, sem_m, sem_o):
    wid = lax.axis_index("s") * _NC + lax.axis_index("c")
    base = wid * _PPW

    def start_in(j, slot):
        cu = pltpu.async_copy(upd_hbm.at[base + j], updv.at[slot], sem_u)
        cm = pltpu.async_copy(msk_hbm.at[base + j], mskv.at[slot], sem_m)
        return cu, cm

    pend = start_in(0, 0)
    out_dma = None
    for j in range(_PPW):
        slot = j % 2
        nxt = start_in(j + 1, 1 - slot) if j + 1 < _PPW else None
        if out_dma is not None:
            out_dma.wait()

        def zbody(i, carry):
            for k in range(8):
                plane[pl.ds(i * 128 + k * 16, 16)] = jnp.zeros((16,), jnp.float32)
            return carry

        lax.fori_loop(0, _R // 128, zbody, 0)

        cu, cm = pend
        cu.wait()
        cm.wait()

        def abody(i, carry):
            m = mskv[slot, pl.ds(i * _LANES, _LANES)]
            r = lax.div(m, jnp.full((_LANES,), _C, jnp.int32))
            v = updv[slot, pl.ds(i * _LANES, _LANES)]
            plsc.addupdate_scatter(plane, [r], v)
            return carry

        lax.fori_loop(0, _NPIX // _LANES, abody, 0)

        out_dma = pltpu.async_copy(plane, out_hbm.at[base + j], sem_o)
        pend = nxt
    out_dma.wait()


@jax.jit
def _unpool(u2, m2):
    mesh = plsc.VectorSubcoreMesh(core_axis_name="c", subcore_axis_name="s")
    return pl.kernel(
        _unpool_body,
        mesh=mesh,
        compiler_params=pltpu.CompilerParams(needs_layout_passes=False),
        out_type=jax.ShapeDtypeStruct((_NPLANES, _R), jnp.float32),
        scratch_types=[
            pltpu.VMEM((_R,), jnp.float32),
            pltpu.VMEM((2, _NPIX), jnp.float32),
            pltpu.VMEM((2, _NPIX), jnp.int32),
            pltpu.SemaphoreType.DMA,
            pltpu.SemaphoreType.DMA,
            pltpu.SemaphoreType.DMA,
        ],
    )(u2, m2)


def kernel(updates, mask):
    B, H, W, C = updates.shape
    u2 = updates.reshape(B, H * W, C).transpose(0, 2, 1).reshape(B * C, H * W)
    m2 = mask.astype(jnp.int32).reshape(B, H * W, C).transpose(0, 2, 1).reshape(B * C, H * W)
    out_t = _unpool(u2, m2)
    out = out_t.reshape(B, C, _OH * _OW).transpose(0, 2, 1)
    return out.reshape(B, _OH, _OW, C)


# speedup vs baseline: 34.1548x; 1.6369x over previous
import functools

import jax
import jax.numpy as jnp
from jax import lax
from jax.experimental import pallas as pl
from jax.experimental.pallas import tpu as pltpu
from jax.experimental.pallas import tpu_sc as plsc

_B, _H, _W, _C = 8, 112, 112, 96
_UP = (2, 2)
_OH, _OW = _H * _UP[0], _W * _UP[1]
_R = _OH * _OW
_NPIX = _H * _W
_NPLANES = _B * _C
_NC, _NS = 2, 16
_NW = _NC * _NS
_PPW = _NPLANES // _NW
_LANES = 16
_AUNROLL = 4


def _unpool_body(upd_hbm, msk_hbm, out_hbm, plane, updv, mskv, sem_u, sem_m, sem_o):
    wid = lax.axis_index("s") * _NC + lax.axis_index("c")
    base = wid * _PPW

    def start_in(j, slot):
        cu = pltpu.async_copy(upd_hbm.at[base + j], updv.at[slot], sem_u)
        cm = pltpu.async_copy(msk_hbm.at[base + j], mskv.at[slot], sem_m)
        return cu, cm

    pend = start_in(0, 0)
    out_dma = None
    for j in range(_PPW):
        slot = j % 2
        nxt = start_in(j + 1, 1 - slot) if j + 1 < _PPW else None
        if out_dma is not None:
            out_dma.wait()

        def zbody(i, carry):
            for k in range(8):
                plane[pl.ds(i * 128 + k * 16, 16)] = jnp.zeros((16,), jnp.float32)
            return carry

        lax.fori_loop(0, _R // 128, zbody, 0)

        cu, cm = pend
        cu.wait()
        cm.wait()

        def abody(i, carry):
            for k in range(_AUNROLL):
                off = (i * _AUNROLL + k) * _LANES
                m = mskv[slot, pl.ds(off, _LANES)]
                x = lax.shift_right_logical(m, jnp.full((_LANES,), 5, jnp.int32))
                xf = x.astype(jnp.float32)
                q = (xf * jnp.float32(1.0 / 3.0)).astype(jnp.int32)
                rem = x - q * 3
                r = q + jnp.where(rem >= 3, 1, 0) - jnp.where(rem < 0, 1, 0)
                v = updv[slot, pl.ds(off, _LANES)]
                plsc.addupdate_scatter(plane, [r], v)
            return carry

        lax.fori_loop(0, _NPIX // (_LANES * _AUNROLL), abody, 0)

        out_dma = pltpu.async_copy(plane, out_hbm.at[base + j], sem_o)
        pend = nxt
    out_dma.wait()


@jax.jit
def _unpool(u2, m2):
    mesh = plsc.VectorSubcoreMesh(core_axis_name="c", subcore_axis_name="s")
    return pl.kernel(
        _unpool_body,
        mesh=mesh,
        compiler_params=pltpu.CompilerParams(needs_layout_passes=False),
        out_type=jax.ShapeDtypeStruct((_NPLANES, _R), jnp.float32),
        scratch_types=[
            pltpu.VMEM((_R,), jnp.float32),
            pltpu.VMEM((2, _NPIX), jnp.float32),
            pltpu.VMEM((2, _NPIX), jnp.int32),
            pltpu.SemaphoreType.DMA,
            pltpu.SemaphoreType.DMA,
            pltpu.SemaphoreType.DMA,
        ],
    )(u2, m2)


def kernel(updates, mask):
    B, H, W, C = updates.shape
    u2 = updates.reshape(B, H * W, C).transpose(0, 2, 1).reshape(B * C, H * W)
    m2 = mask.astype(jnp.int32).reshape(B, H * W, C).transpose(0, 2, 1).reshape(B * C, H * W)
    out_t = _unpool(u2, m2)
    out = out_t.reshape(B, C, _OH * _OW).transpose(0, 2, 1)
    return out.reshape(B, _OH, _OW, C)


# speedup vs baseline: 35.9257x; 1.0518x over previous
import jax
import jax.numpy as jnp
import numpy as np
from jax import lax
from jax.experimental import pallas as pl
from jax.experimental.pallas import tpu as pltpu
from jax.experimental.pallas import tpu_sc as plsc

_B, _H, _W, _C = 8, 112, 112, 96
_UP = (2, 2)
_OH, _OW = _H * _UP[0], _W * _UP[1]
_R = _OH * _OW
_NPIX = _H * _W
_HP = _NPIX // 2
_NPLANES = _B * _C
_NC, _NS = 2, 16
_NW = _NC * _NS
_PPW = _NPLANES // _NW
_LANES = 16
_AUNROLL = 4
_RECIP = np.float32(1.0 / _C)


def _unpool_body(upd_hbm, msk_hbm, out_hbm, plane0, plane1, updv, mskv,
                 sem_u, sem_m, sem_o):
    planes = (plane0, plane1)
    wid = lax.axis_index("s") * _NC + lax.axis_index("c")
    base = wid * _PPW

    def start_in(g, slot):
        cu = pltpu.async_copy(upd_hbm.at[2 * base + g], updv.at[slot], sem_u)
        cm = pltpu.async_copy(msk_hbm.at[2 * base + g], mskv.at[slot], sem_m)
        return cu, cm

    pend = start_in(0, 0)
    flush = [None, None]
    for j in range(_PPW):
        pslot = j % 2
        plane = planes[pslot]
        if flush[pslot] is not None:
            flush[pslot].wait()

        def zbody(i, carry):
            for k in range(8):
                plane[pl.ds(i * 128 + k * 16, 16)] = jnp.zeros(
                    (16,), jnp.float32)
            return carry

        lax.fori_loop(0, _R // 128, zbody, 0)

        for h in range(2):
            g = 2 * j + h
            islot = g % 2
            nxt = start_in(g + 1, 1 - islot) if g + 1 < 2 * _PPW else None
            cu, cm = pend
            cu.wait()
            cm.wait()

            def abody(i, carry):
                for k in range(_AUNROLL):
                    off = (i * _AUNROLL + k) * _LANES
                    m = mskv[islot, pl.ds(off, _LANES)]
                    r = (m.astype(jnp.float32) * _RECIP).astype(jnp.int32)
                    v = updv[islot, pl.ds(off, _LANES)]
                    plsc.addupdate_scatter(plane, [r], v)
                return carry

            lax.fori_loop(0, _HP // (_LANES * _AUNROLL), abody, 0)
            pend = nxt

        flush[pslot] = pltpu.async_copy(
            plane, out_hbm.at[base + j], sem_o)
    for f in flush:
        f.wait()


@jax.jit
def _unpool(u2, m2):
    mesh = plsc.VectorSubcoreMesh(core_axis_name="c", subcore_axis_name="s")
    return pl.kernel(
        _unpool_body,
        mesh=mesh,
        compiler_params=pltpu.CompilerParams(needs_layout_passes=False),
        out_type=jax.ShapeDtypeStruct((_NPLANES, _R), jnp.float32),
        scratch_types=[
            pltpu.VMEM((_R,), jnp.float32),
            pltpu.VMEM((_R,), jnp.float32),
            pltpu.VMEM((2, _HP), jnp.float32),
            pltpu.VMEM((2, _HP), jnp.int32),
            pltpu.SemaphoreType.DMA,
            pltpu.SemaphoreType.DMA,
            pltpu.SemaphoreType.DMA,
        ],
    )(u2, m2)


def kernel(updates, mask):
    B, H, W, C = updates.shape
    u2 = updates.reshape(B, H * W, C).transpose(0, 2, 1).reshape(B * C * 2, _HP)
    m2 = mask.astype(jnp.int32).reshape(B, H * W, C).transpose(0, 2, 1).reshape(B * C * 2, _HP)
    out_t = _unpool(u2, m2)
    out = out_t.reshape(B, C, _OH * _OW).transpose(0, 2, 1)
    return out.reshape(B, _OH, _OW, C)


# speedup vs baseline: 39.7287x; 1.1059x over previous
import jax
import jax.numpy as jnp
import numpy as np
from jax import lax
from jax.experimental import pallas as pl
from jax.experimental.pallas import tpu as pltpu
from jax.experimental.pallas import tpu_sc as plsc

_B, _H, _W, _C = 8, 112, 112, 96
_UP = (2, 2)
_OH, _OW = _H * _UP[0], _W * _UP[1]
_R = _OH * _OW
_NPIX = _H * _W
_HP = _NPIX // 2
_NPLANES = _B * _C
_NC, _NS = 2, 16
_NW = _NC * _NS
_PPW = _NPLANES // _NW
_LANES = 16
_AUNROLL = 4
_RECIP = np.float32(1.0 / _C)


def _unpool_body(upd_hbm, msk_hbm, out_hbm, plane0, plane1, updv, mskv,
                 sem_u, sem_m, sem_o):
    planes = (plane0, plane1)
    wid = lax.axis_index("s") * _NC + lax.axis_index("c")
    base = wid * _PPW

    def start_in(g, slot):
        cu = pltpu.async_copy(upd_hbm.at[2 * base + g], updv.at[slot], sem_u)
        cm = pltpu.async_copy(msk_hbm.at[2 * base + g], mskv.at[slot], sem_m)
        return cu, cm

    pend = start_in(0, 0)
    flush = [None, None]
    for j in range(_PPW):
        pslot = j % 2
        plane = planes[pslot]
        if flush[pslot] is not None:
            flush[pslot].wait()

        def zbody(i, carry):
            for k in range(8):
                plane[pl.ds(i * 128 + k * 16, 16)] = jnp.zeros(
                    (16,), jnp.float32)
            return carry

        lax.fori_loop(0, _R // 128, zbody, 0)

        for h in range(2):
            g = 2 * j + h
            islot = g % 2
            nxt = start_in(g + 1, 1 - islot) if g + 1 < 2 * _PPW else None
            cu, cm = pend
            cu.wait()
            cm.wait()

            def abody(i, carry):
                for k in range(_AUNROLL):
                    off = (i * _AUNROLL + k) * _LANES
                    r = mskv[islot, pl.ds(off, _LANES)]
                    v = updv[islot, pl.ds(off, _LANES)]
                    plsc.addupdate_scatter(plane, [r], v)
                return carry

            lax.fori_loop(0, _HP // (_LANES * _AUNROLL), abody, 0)
            pend = nxt

        flush[pslot] = pltpu.async_copy(
            plane, out_hbm.at[base + j], sem_o)
    for f in flush:
        f.wait()


@jax.jit
def _unpool(u2, m2):
    mesh = plsc.VectorSubcoreMesh(core_axis_name="c", subcore_axis_name="s")
    return pl.kernel(
        _unpool_body,
        mesh=mesh,
        compiler_params=pltpu.CompilerParams(needs_layout_passes=False),
        out_type=jax.ShapeDtypeStruct((_NPLANES, _R), jnp.float32),
        scratch_types=[
            pltpu.VMEM((_R,), jnp.float32),
            pltpu.VMEM((_R,), jnp.float32),
            pltpu.VMEM((2, _HP), jnp.float32),
            pltpu.VMEM((2, _HP), jnp.int32),
            pltpu.SemaphoreType.DMA,
            pltpu.SemaphoreType.DMA,
            pltpu.SemaphoreType.DMA,
        ],
    )(u2, m2)


def kernel(updates, mask):
    B, H, W, C = updates.shape
    u2 = updates.reshape(B, H * W, C).transpose(0, 2, 1).reshape(B * C * 2, _HP)
    r = lax.div(mask.astype(jnp.int32), jnp.int32(C))
    m2 = r.reshape(B, H * W, C).transpose(0, 2, 1).reshape(B * C * 2, _HP)
    out_t = _unpool(u2, m2)
    out = out_t.reshape(B, C, _OH * _OW).transpose(0, 2, 1)
    return out.reshape(B, _OH, _OW, C)
